# Initial kernel scaffold; baseline (speedup 1.0000x reference)
import jax
import jax.numpy as jnp
from jax.experimental import pallas as pl


def kernel(node_attr, edge_attr, edge_index, params):
    raise NotImplementedError("write your pallas kernel here")


# speedup vs baseline: 1.0002x; 1.0002x over previous
import jax
import jax.numpy as jnp
from jax.experimental import pallas as pl


def _final_pallas(g, w, b):
    def body(g_ref, w_ref, b_ref, o_ref):
        acc = jnp.sum(g_ref[...] * w_ref[...]) + jnp.sum(b_ref[...])
        o_ref[...] = jnp.broadcast_to(jnp.maximum(acc, 0.0), (1, 1))

    return pl.pallas_call(
        body,
        out_shape=jax.ShapeDtypeStruct((1, 1), jnp.float32),
    )(g.reshape(1, 64), w.reshape(1, 64), b.reshape(1, 1))


def _pna(h, e, src, dst, deg, n, p, residual):
    f = jnp.concatenate([h[src], h[dst], e], axis=-1)
    msg = f @ p["M_w"].T + p["M_b"]
    s = jax.ops.segment_sum(msg, dst, num_segments=n)
    s2 = jax.ops.segment_sum(msg * msg, dst, num_segments=n)
    mx = jax.ops.segment_max(msg, dst, num_segments=n)
    mx = jnp.where(jnp.isfinite(mx), mx, 0.0)
    degc = jnp.maximum(deg, 1.0)[:, None]
    mean = s / degc
    var = jax.nn.relu(s2 / degc - mean * mean)
    std = jnp.sqrt(var + 1e-30)
    h_neigh = jnp.concatenate([mean, mx, s, std], axis=-1)
    hh = jnp.concatenate([h, h_neigh], axis=-1) @ p["U_w"].T + p["U_b"]
    hh = hh / n
    m = jnp.mean(hh, axis=0)
    v = jnp.var(hh, axis=0)
    hh = (hh - m) / jnp.sqrt(v + 1e-5) * p["bn_g"] + p["bn_b"]
    out = hh @ p["mix_w"].T + p["mix_b"]
    out = jnp.where(out >= 0, out, 0.01 * out)
    if residual:
        out = out + h
    return out


def kernel(node_attr, edge_attr, edge_index, params):
    src = edge_index[0]
    dst = edge_index[1]
    n = node_attr.shape[0]
    deg = jax.ops.segment_sum(jnp.ones((src.shape[0],), jnp.float32), dst, num_segments=n)
    ns = node_attr[:, :-1]
    ns_s = node_attr[:, -1]
    h = jax.nn.relu(_pna(ns, edge_attr, src, dst, deg, n, params["p1"], False))
    h = jax.nn.relu(_pna(h, edge_attr, src, dst, deg, n, params["p2"], True))
    h = jax.nn.relu(_pna(h, edge_attr, src, dst, deg, n, params["p3"], False))
    t = jnp.concatenate([h, jnp.tile(ns_s[:, None], (1, 4))], axis=-1)
    t = jax.nn.relu(_pna(t, edge_attr, src, dst, deg, n, params["p01"], False))
    t = jax.nn.relu(_pna(t, edge_attr, src, dst, deg, n, params["p02"], True))
    g = jnp.max(t, axis=0)
    logits = _final_pallas(g, params["conv_w"], params["conv_b"])
    return logits.reshape(1, 1, 1)


# speedup vs baseline: 1.5092x; 1.5089x over previous
import functools

import jax
import jax.numpy as jnp
from jax import lax
from jax.experimental import pallas as pl
from jax.experimental.pallas import tpu as pltpu
from jax.experimental.pallas import tpu_sc as plsc

N_NODES = 50000
N_EDGES = 800000
NT = 32
NPB = 256
NB = (N_NODES + NPB - 1) // NPB
NPAD = NB * NPB
EPT = N_EDGES // NT
E_ALLOC = N_EDGES + NT * NB * 16 + 1024
DUMP = E_ALLOC - 1
ROWB = 1000
NROWB = N_NODES // ROWB

_SC_PARAMS = pltpu.CompilerParams(
    needs_layout_passes=False, use_tc_tiling_on_sc=False)


def _mesh():
    return plsc.VectorSubcoreMesh(core_axis_name="c", subcore_axis_name="s")


def _iota16():
    return lax.broadcasted_iota(jnp.int32, (16,), 0)


def _full16(v):
    return jnp.full((16,), v, jnp.int32)


def _lane(vec, i):
    return vec.at[_full16(i)].get(mode="promise_in_bounds")


def _extract(vec, lane_idx):
    return jnp.sum(jnp.where(_iota16() == lane_idx, vec, 0))


def _sc_hist(dst):
    @functools.partial(
        pl.kernel,
        out_type=jax.ShapeDtypeStruct((NT * 256,), jnp.int32),
        mesh=_mesh(),
        compiler_params=_SC_PARAMS,
        scratch_types=[
            pltpu.VMEM((EPT + 32,), jnp.int32),
            pltpu.VMEM((256,), jnp.int32),
            pltpu.SMEM((256,), jnp.int32),
        ],
    )
    def k(dst_h, out_h, dbuf, histv, cs):
        IOTA = _iota16()
        wid = lax.axis_index("s") * 2 + lax.axis_index("c")
        pltpu.sync_copy(dst_h.at[pl.ds(pl.multiple_of(wid * EPT, 8), EPT)], dbuf.at[pl.ds(0, EPT)])

        def zero(i, _):
            cs[i] = jnp.int32(0)
            return 0

        lax.fori_loop(0, 256, zero, 0)

        def grp(g, _):
            d16 = plsc.load_gather(dbuf, [_full16(0) + g * 16 + IOTA])
            b16 = lax.shift_right_logical(d16, 8)
            for i in range(16):
                bi = b16[i]
                cs[bi] = cs[bi] + 1
            return 0

        lax.fori_loop(0, EPT // 16, grp, 0)
        nt_ = EPT % 16
        if nt_:
            d16 = plsc.load_gather(dbuf, [_full16((EPT // 16) * 16) + IOTA])
            b16 = lax.shift_right_logical(d16, 8)
            for i in range(nt_):
                bi = b16[i]
                cs[bi] = cs[bi] + 1

        def wb(g, _):
            v = jnp.zeros((16,), jnp.int32)
            for i in range(16):
                v = jnp.where(IOTA == i, cs[g * 16 + i], v)
            plsc.store_scatter(histv, [_full16(0) + g * 16 + IOTA], v)
            return 0

        lax.fori_loop(0, 16, wb, 0)
        pltpu.sync_copy(histv, out_h.at[pl.ds(pl.multiple_of(wid * 256, 8), 256)])

    return k(dst)


def _tc_offsets(hist2d):
    def body(h_ref, cur_ref, bs_ref, bc_ref):
        h = h_ref[...].astype(jnp.float32)
        hp = jnp.floor((h + 15.0) * (1.0 / 16.0)) * 16.0
        ti = lax.broadcasted_iota(jnp.int32, (NT, NT), 0)
        tj = lax.broadcasted_iota(jnp.int32, (NT, NT), 1)
        sl = jnp.where(tj < ti, 1.0, 0.0).astype(jnp.float32)
        tilepfx = jax.lax.dot_general(sl, hp, (((1,), (0,)), ((), ())),
                                      preferred_element_type=jnp.float32)
        s1 = jnp.sum(hp, axis=0, keepdims=True)
        bi_ = lax.broadcasted_iota(jnp.int32, (256, 256), 0)
        bj_ = lax.broadcasted_iota(jnp.int32, (256, 256), 1)
        su = jnp.where(bi_ < bj_, 1.0, 0.0).astype(jnp.float32)
        bb = jax.lax.dot_general(s1, su, (((1,), (0,)), ((), ())),
                                 preferred_element_type=jnp.float32)
        cur_ref[...] = (bb + tilepfx).astype(jnp.int32)
        bs_ref[...] = jnp.broadcast_to(bb, (8, 256)).astype(jnp.int32)
        bc_ref[...] = jnp.broadcast_to(s1, (8, 256)).astype(jnp.int32)

    return pl.pallas_call(
        body,
        out_shape=(
            jax.ShapeDtypeStruct((NT, 256), jnp.int32),
            jax.ShapeDtypeStruct((8, 256), jnp.int32),
            jax.ShapeDtypeStruct((8, 256), jnp.int32),
        ),
    )(hist2d)


EC = 3200
NFULL = EPT // EC
ETAIL = EPT - NFULL * EC
TG = ETAIL // 16
TREM = ETAIL - TG * 16


def _sc_scatter(dst, src, e0, e1, cur0):
    out_i32 = jax.ShapeDtypeStruct((E_ALLOC,), jnp.int32)
    out_f32 = jax.ShapeDtypeStruct((E_ALLOC,), jnp.float32)

    @functools.partial(
        pl.kernel,
        out_type=(out_i32, out_i32, out_f32, out_f32),
        mesh=_mesh(),
        compiler_params=_SC_PARAMS,
        scratch_types=[
            pltpu.VMEM((EC,), jnp.int32),
            pltpu.VMEM((EC,), jnp.int32),
            pltpu.VMEM((EC,), jnp.float32),
            pltpu.VMEM((EC,), jnp.float32),
            pltpu.VMEM((EC,), jnp.int32),
            pltpu.VMEM((25, 128), jnp.int32),
            pltpu.VMEM((256,), jnp.int32),
            pltpu.VMEM((32, 128), jnp.int32),
            pltpu.VMEM((256,), jnp.int32),
            pltpu.VMEM((128,), jnp.float32),
            pltpu.SMEM((256,), jnp.int32),
            pltpu.SemaphoreType.DMA,
        ],
    )
    def k(dst_h, src_h, e0_h, e1_h, cur0_h,
          srcp_h, dlocp_h, e0p_h, e1p_h,
          dstg, srcg, e0g, e1g, dlg, posb, curv, padpos, padi, padf, cs, sem):
        IOTA = _iota16()
        wid = lax.axis_index("s") * 2 + lax.axis_index("c")
        base_e = wid * EPT
        pltpu.sync_copy(cur0_h.at[pl.ds(pl.multiple_of(wid * 256, 8), 256)], curv)

        def stage_cur(g, _):
            v = plsc.load_gather(curv, [_full16(0) + g * 16 + IOTA])
            for i in range(16):
                cs[g * 16 + i] = v[i]
            return 0

        lax.fori_loop(0, 16, stage_cur, 0)

        def do_group(gbase, nlanes):
            d16 = plsc.load_gather(dstg, [_full16(0) + gbase + IOTA])
            b16 = lax.shift_right_logical(d16, 8)
            dl16 = lax.bitwise_and(d16, jnp.int32(255))
            plsc.store_scatter(dlg, [_full16(0) + gbase + IOTA], dl16)
            posv = _full16(DUMP)
            for i in range(nlanes):
                bi = b16[i]
                p = cs[bi]
                cs[bi] = p + 1
                posv = jnp.where(IOTA == i, p, posv)
            prow = lax.div(gbase, jnp.int32(128))
            pcol = lax.rem(gbase, jnp.int32(128))
            plsc.store_scatter(posb, [_full16(0) + prow, pcol + IOTA], posv)

        def flush():
            for r in range(25):
                cps = []
                for stg, outh in ((srcg, srcp_h), (dlg, dlocp_h),
                                  (e0g, e0p_h), (e1g, e1p_h)):
                    cps.append(pltpu.async_copy(
                        stg.at[pl.ds(r * 128, 128)],
                        outh.at[posb.at[r]], sem))
                for cp in cps:
                    cp.wait()

        def chunk(cidx, _):
            off = pl.multiple_of(base_e + cidx * EC, 8)
            pltpu.sync_copy(dst_h.at[pl.ds(off, EC)], dstg)
            pltpu.sync_copy(src_h.at[pl.ds(off, EC)], srcg)
            pltpu.sync_copy(e0_h.at[pl.ds(off, EC)], e0g)
            pltpu.sync_copy(e1_h.at[pl.ds(off, EC)], e1g)

            def grp(g, _):
                do_group(g * 16, 16)
                return 0

            lax.fori_loop(0, EC // 16, grp, 0)
            flush()
            return 0

        lax.fori_loop(0, NFULL, chunk, 0)

        off = pl.multiple_of(base_e + NFULL * EC, 8)
        pltpu.sync_copy(dst_h.at[pl.ds(off, ETAIL)], dstg.at[pl.ds(0, ETAIL)])
        pltpu.sync_copy(src_h.at[pl.ds(off, ETAIL)], srcg.at[pl.ds(0, ETAIL)])
        pltpu.sync_copy(e0_h.at[pl.ds(off, ETAIL)], e0g.at[pl.ds(0, ETAIL)])
        pltpu.sync_copy(e1_h.at[pl.ds(off, ETAIL)], e1g.at[pl.ds(0, ETAIL)])
        dv = _full16(DUMP)

        def pfill(r, _):
            for c in range(8):
                plsc.store_scatter(posb, [_full16(0) + r, c * 16 + IOTA], dv)
            return 0

        lax.fori_loop(0, 25, pfill, 0)

        def grp_t(g, _):
            do_group(g * 16, 16)
            return 0

        lax.fori_loop(0, TG, grp_t, 0)
        if TREM:
            do_group(TG * 16, TREM)
        flush()

        zi = jnp.zeros((16,), jnp.int32)
        zf = jnp.zeros((16,), jnp.float32)
        for c in range(16):
            plsc.store_scatter(padi, [_full16(c * 16) + IOTA],
                               zi if c < 8 else _full16(256))
        for c in range(8):
            plsc.store_scatter(padf, [_full16(c * 16) + IOTA], zf)

        def padb(b, _):
            p0 = cs[b]
            npad = lax.bitwise_and(-p0, jnp.int32(15))
            pv = jnp.where(IOTA < npad, p0 + IOTA, _full16(DUMP))
            prow = lax.shift_right_logical(b, 3)
            pcol = lax.rem(b, jnp.int32(8)) * 16
            plsc.store_scatter(padpos, [_full16(0) + prow, pcol + IOTA], pv)
            return 0

        lax.fori_loop(0, 256, padb, 0)
        for r in range(32):
            cps = [
                pltpu.async_copy(padi.at[pl.ds(0, 128)],
                                 srcp_h.at[padpos.at[r]], sem),
                pltpu.async_copy(padi.at[pl.ds(128, 128)],
                                 dlocp_h.at[padpos.at[r]], sem),
                pltpu.async_copy(padf, e0p_h.at[padpos.at[r]], sem),
                pltpu.async_copy(padf, e1p_h.at[padpos.at[r]], sem),
            ]
            for cp in cps:
                cp.wait()

    return k(dst, src, e0, e1, cur0)


def _sc_agg(Fp, A, srcp, dlocp, e0p, e1p, bstart, bcount, m0, m1):
    NCH = Fp // 16

    @functools.partial(
        pl.kernel,
        out_type=(
            jax.ShapeDtypeStruct((NPAD * Fp,), jnp.float32),
            jax.ShapeDtypeStruct((NPAD * Fp,), jnp.float32),
            jax.ShapeDtypeStruct((NPAD * Fp,), jnp.float32),
            jax.ShapeDtypeStruct((NPAD,), jnp.int32),
        ),
        mesh=_mesh(),
        compiler_params=_SC_PARAMS,
        scratch_types=[
            pltpu.VMEM((257 * Fp,), jnp.float32),
            pltpu.VMEM((257 * Fp,), jnp.float32),
            pltpu.VMEM((257 * Fp,), jnp.float32),
            pltpu.VMEM((256,), jnp.int32),
            pltpu.VMEM((128,), jnp.int32),
            pltpu.VMEM((128,), jnp.int32),
            pltpu.VMEM((128,), jnp.float32),
            pltpu.VMEM((128,), jnp.float32),
            pltpu.VMEM((128, Fp), jnp.float32),
            pltpu.VMEM((256,), jnp.int32),
            pltpu.VMEM((256,), jnp.int32),
            pltpu.VMEM((Fp,), jnp.float32),
            pltpu.VMEM((Fp,), jnp.float32),
            pltpu.SMEM((272,), jnp.int32),
            pltpu.SemaphoreType.DMA,
        ],
    )
    def k(A_h, srcp_h, dlocp_h, e0p_h, e1p_h, bs_h, bc_h, m0_h, m1_h,
          sum_h, sq_h, mx_h, cnt_h,
          accs, accq, accm, cntv, sbuf, dlbuf, e0buf, e1buf, rows,
          bsv, bcv, m0v, m1v, cnts, sem):
        IOTA = _iota16()
        wid = lax.axis_index("s") * 2 + lax.axis_index("c")
        pltpu.sync_copy(bs_h, bsv)
        pltpu.sync_copy(bc_h, bcv)
        pltpu.sync_copy(m0_h, m0v)
        pltpu.sync_copy(m1_h, m1v)
        m0c = [plsc.load_gather(m0v, [IOTA + c * 16]) for c in range(NCH)]
        m1c = [plsc.load_gather(m1v, [IOTA + c * 16]) for c in range(NCH)]

        def bucket(kk, _):
            b = wid + kk * 32

            @pl.when(b < NB)
            def _():
                zf = jnp.zeros((16,), jnp.float32)
                ninf = jnp.full((16,), -3.0e38, jnp.float32)

                def zrow(r, _):
                    rb = _full16(0) + r * Fp
                    for c in range(NCH):
                        fi = rb + (IOTA + c * 16)
                        plsc.store_scatter(accs, [fi], zf)
                        plsc.store_scatter(accq, [fi], zf)
                        plsc.store_scatter(accm, [fi], ninf)
                    return 0

                lax.fori_loop(0, 257, zrow, 0)

                def czero(i, _):
                    cnts[i] = jnp.int32(0)
                    return 0

                lax.fori_loop(0, 272, czero, 0)

                cbase = lax.shift_left(lax.shift_right_logical(b, 4), 4)
                lane = lax.bitwise_and(b, jnp.int32(15))
                start = _extract(
                    plsc.load_gather(bsv, [_full16(0) + cbase + IOTA]), lane)
                cnt = _extract(
                    plsc.load_gather(bcv, [_full16(0) + cbase + IOTA]), lane)
                ngroups = lax.div(cnt + 127, jnp.int32(128))

                def group(g, _):
                    off = pl.multiple_of(start + g * 128, 8)
                    pltpu.sync_copy(srcp_h.at[pl.ds(off, 128)], sbuf)
                    pltpu.sync_copy(dlocp_h.at[pl.ds(off, 128)], dlbuf)
                    pltpu.sync_copy(e0p_h.at[pl.ds(off, 128)], e0buf)
                    pltpu.sync_copy(e1p_h.at[pl.ds(off, 128)], e1buf)
                    vg = cnt - g * 128

                    @pl.when(vg < 128)
                    def _():
                        for jb in range(8):
                            j16 = _full16(jb * 16) + IOTA
                            sv = plsc.load_gather(sbuf, [j16])
                            sv = jnp.where(j16 < vg, sv, 0)
                            plsc.store_scatter(sbuf, [j16], sv)

                    pltpu.async_copy(A_h.at[sbuf], rows, sem).wait()
                    nsub = jnp.minimum(jnp.int32(8),
                                       lax.div(vg, jnp.int32(16)))

                    def sub(s, _):
                        sb = s * 16
                        dl16 = plsc.load_gather(dlbuf,
                                                [_full16(0) + sb + IOTA])
                        e016 = plsc.load_gather(e0buf, [_full16(0) + sb + IOTA])
                        e116 = plsc.load_gather(e1buf, [_full16(0) + sb + IOTA])
                        for i in range(16):
                            di = dl16[i]
                            cnts[di] = cnts[di] + 1
                            dli = _lane(dl16, i)
                            e0i = e016.at[_full16(i)].get(
                                mode="promise_in_bounds")
                            e1i = e116.at[_full16(i)].get(
                                mode="promise_in_bounds")
                            rb = dli * Fp
                            for c in range(NCH):
                                row = plsc.load_gather(
                                    rows,
                                    [_full16(0) + sb + i, IOTA + c * 16])
                                p = row + e0i * m0c[c] + e1i * m1c[c]
                                fi = rb + (IOTA + c * 16)
                                plsc.addupdate_scatter(accs, [fi], p)
                                plsc.addupdate_scatter(accq, [fi], p * p)
                                cm = plsc.load_gather(accm, [fi])
                                plsc.store_scatter(accm, [fi],
                                                   jnp.maximum(cm, p))
                        return 0

                    lax.fori_loop(0, nsub, sub, 0)
                    return 0

                lax.fori_loop(0, ngroups, group, 0)

                pltpu.sync_copy(accs.at[pl.ds(0, 256 * Fp)],
                                sum_h.at[pl.ds(pl.multiple_of(b * (256 * Fp), 8), 256 * Fp)])
                pltpu.sync_copy(accq.at[pl.ds(0, 256 * Fp)],
                                sq_h.at[pl.ds(pl.multiple_of(b * (256 * Fp), 8), 256 * Fp)])
                pltpu.sync_copy(accm.at[pl.ds(0, 256 * Fp)],
                                mx_h.at[pl.ds(pl.multiple_of(b * (256 * Fp), 8), 256 * Fp)])

                def cwb(g, _):
                    v = jnp.zeros((16,), jnp.int32)
                    for i in range(16):
                        v = jnp.where(IOTA == i, cnts[g * 16 + i], v)
                    plsc.store_scatter(cntv, [_full16(0) + g * 16 + IOTA], v)
                    return 0

                lax.fori_loop(0, 16, cwb, 0)
                pltpu.sync_copy(cntv, cnt_h.at[pl.ds(pl.multiple_of(b * 256, 8), 256)])

            return 0

        lax.fori_loop(0, 7, bucket, 0)

    return k(A, srcp, dlocp, e0p, e1p, bstart, bcount, m0, m1)


def _dot(a, b):
    return jax.lax.dot_general(a, b, (((1,), (0,)), ((), ())),
                               preferred_element_type=jnp.float32)


def _tc_pre(h, W1, b1, W2, Fp):
    Fin = h.shape[1]

    def body(h_ref, w1_ref, b1_ref, w2_ref, a_ref, b_ref):
        hb = h_ref[...]
        a_ref[...] = _dot(hb, w1_ref[...]) + b1_ref[...]
        b_ref[...] = _dot(hb, w2_ref[...])

    return pl.pallas_call(
        body,
        grid=(NROWB,),
        in_specs=[
            pl.BlockSpec((ROWB, Fin), lambda i: (i, 0)),
            pl.BlockSpec((Fin, Fp), lambda i: (0, 0)),
            pl.BlockSpec((1, Fp), lambda i: (0, 0)),
            pl.BlockSpec((Fin, Fp), lambda i: (0, 0)),
        ],
        out_specs=(
            pl.BlockSpec((ROWB, Fp), lambda i: (i, 0)),
            pl.BlockSpec((ROWB, Fp), lambda i: (i, 0)),
        ),
        out_shape=(
            jax.ShapeDtypeStruct((N_NODES, Fp), jnp.float32),
            jax.ShapeDtypeStruct((N_NODES, Fp), jnp.float32),
        ),
    )(h, W1, b1, W2)


def _tc_post1(h, Bm, SUM, SQ, MX, deg2d, Uw_h, Uw_mean, Uw_mx, Uw_s, Uw_std,
              Ub, O):
    Fin = h.shape[1]
    Fp = Bm.shape[1]

    def body(h_ref, b_ref, sum_ref, sq_ref, mx_ref, deg_ref,
             uh_ref, um_ref, ux_ref, us_ref, ud_ref, ub_ref,
             hh_ref, st_ref, sacc):
        i = pl.program_id(0)

        @pl.when(i == 0)
        def _():
            sacc[...] = jnp.zeros((8, O), jnp.float32)

        hb = h_ref[...]
        B = b_ref[...]
        SA = sum_ref[...]
        SQb = sq_ref[...]
        MXb = mx_ref[...]
        deg = deg_ref[...]
        degc = jnp.maximum(deg, 1.0)
        s = SA + deg * B
        s2 = SQb + 2.0 * B * SA + deg * B * B
        mx = jnp.where(deg > 0, B + MXb, 0.0)
        mean = s / degc
        var = jnp.maximum(s2 / degc - mean * mean, 0.0)
        std = jnp.sqrt(var + 1e-30)
        hh = (_dot(hb, uh_ref[...]) + _dot(mean, um_ref[...])
              + _dot(mx, ux_ref[...]) + _dot(s, us_ref[...])
              + _dot(std, ud_ref[...]) + ub_ref[...])
        hh = hh * (1.0 / N_NODES)
        hh_ref[...] = hh
        sacc[0:1, :] = sacc[0:1, :] + jnp.sum(hh, axis=0, keepdims=True)
        sacc[1:2, :] = sacc[1:2, :] + jnp.sum(hh * hh, axis=0, keepdims=True)

        @pl.when(i == NROWB - 1)
        def _():
            st_ref[...] = sacc[...]

    return pl.pallas_call(
        body,
        grid=(NROWB,),
        in_specs=[
            pl.BlockSpec((ROWB, Fin), lambda i: (i, 0)),
            pl.BlockSpec((ROWB, Fp), lambda i: (i, 0)),
            pl.BlockSpec((ROWB, Fp), lambda i: (i, 0)),
            pl.BlockSpec((ROWB, Fp), lambda i: (i, 0)),
            pl.BlockSpec((ROWB, Fp), lambda i: (i, 0)),
            pl.BlockSpec((ROWB, 1), lambda i: (i, 0)),
            pl.BlockSpec((Fin, O), lambda i: (0, 0)),
            pl.BlockSpec((Fp, O), lambda i: (0, 0)),
            pl.BlockSpec((Fp, O), lambda i: (0, 0)),
            pl.BlockSpec((Fp, O), lambda i: (0, 0)),
            pl.BlockSpec((Fp, O), lambda i: (0, 0)),
            pl.BlockSpec((1, O), lambda i: (0, 0)),
        ],
        out_specs=(
            pl.BlockSpec((ROWB, O), lambda i: (i, 0)),
            pl.BlockSpec((8, O), lambda i: (0, 0)),
        ),
        out_shape=(
            jax.ShapeDtypeStruct((N_NODES, O), jnp.float32),
            jax.ShapeDtypeStruct((8, O), jnp.float32),
        ),
        scratch_shapes=[pltpu.VMEM((8, O), jnp.float32)],
    )(h, Bm, SUM, SQ, MX, deg2d, Uw_h, Uw_mean, Uw_mx, Uw_s, Uw_std, Ub)


def _tc_post2(hh, stats, hprev, bn_g, bn_b, mixWT, mixb, O, residual,
              out_width=None, ns_s=None):
    ow = out_width or O
    in_specs = [
        pl.BlockSpec((ROWB, O), lambda i: (i, 0)),
        pl.BlockSpec((8, O), lambda i: (0, 0)),
        pl.BlockSpec((1, O), lambda i: (0, 0)),
        pl.BlockSpec((1, O), lambda i: (0, 0)),
        pl.BlockSpec((O, O), lambda i: (0, 0)),
        pl.BlockSpec((1, O), lambda i: (0, 0)),
    ]
    args = [hh, stats, bn_g, bn_b, mixWT, mixb]
    if residual:
        in_specs.append(pl.BlockSpec((ROWB, O), lambda i: (i, 0)))
        args.append(hprev)
    if ns_s is not None:
        in_specs.append(pl.BlockSpec((ROWB, 1), lambda i: (i, 0)))
        args.append(ns_s)

    def body(*refs):
        hh_ref, st_ref, g_ref, bb_ref, mw_ref, mb_ref = refs[:6]
        rest = list(refs[6:-1])
        o_ref = refs[-1]
        hb = hh_ref[...]
        st = st_ref[...]
        m = st[0:1, :] * (1.0 / N_NODES)
        msq = st[1:2, :] * (1.0 / N_NODES)
        inv = jax.lax.rsqrt(msq - m * m + 1e-5)
        xn = (hb - m) * inv * g_ref[...] + bb_ref[...]
        out = _dot(xn, mw_ref[...]) + mb_ref[...]
        out = jnp.where(out >= 0, out, 0.01 * out)
        if residual:
            out = out + rest.pop(0)[...]
        out = jnp.maximum(out, 0.0)
        if ns_s is not None:
            nss = rest.pop(0)[...]
            out = jnp.concatenate(
                [out, jnp.broadcast_to(nss, (ROWB, 4)),
                 jnp.zeros((ROWB, ow - O - 4), jnp.float32)], axis=1)
        o_ref[...] = out

    return pl.pallas_call(
        body,
        grid=(NROWB,),
        in_specs=in_specs,
        out_specs=pl.BlockSpec((ROWB, ow), lambda i: (i, 0)),
        out_shape=jax.ShapeDtypeStruct((N_NODES, ow), jnp.float32),
    )(*args)


def _tc_post2_final(hh, stats, hprev, bn_g, bn_b, mixWT, mixb, conv_w, conv_b):
    O = 64

    def body(hh_ref, st_ref, g_ref, bb_ref, mw_ref, mb_ref, hp_ref,
             cw_ref, cb_ref, o_ref, gm):
        i = pl.program_id(0)

        @pl.when(i == 0)
        def _():
            gm[...] = jnp.full((8, O), -3.0e38, jnp.float32)

        hb = hh_ref[...]
        st = st_ref[...]
        m = st[0:1, :] * (1.0 / N_NODES)
        msq = st[1:2, :] * (1.0 / N_NODES)
        inv = jax.lax.rsqrt(msq - m * m + 1e-5)
        xn = (hb - m) * inv * g_ref[...] + bb_ref[...]
        out = _dot(xn, mw_ref[...]) + mb_ref[...]
        out = jnp.where(out >= 0, out, 0.01 * out)
        out = jnp.maximum(out + hp_ref[...], 0.0)
        bm = jnp.max(out, axis=0, keepdims=True)
        gm[0:1, :] = jnp.maximum(gm[0:1, :], bm)

        @pl.when(i == NROWB - 1)
        def _():
            val = jnp.sum(gm[0:1, :] * cw_ref[...]) + jnp.sum(cb_ref[...])
            o_ref[...] = jnp.broadcast_to(jnp.maximum(val, 0.0), (1, 1))

    return pl.pallas_call(
        body,
        grid=(NROWB,),
        in_specs=[
            pl.BlockSpec((ROWB, O), lambda i: (i, 0)),
            pl.BlockSpec((8, O), lambda i: (0, 0)),
            pl.BlockSpec((1, O), lambda i: (0, 0)),
            pl.BlockSpec((1, O), lambda i: (0, 0)),
            pl.BlockSpec((O, O), lambda i: (0, 0)),
            pl.BlockSpec((1, O), lambda i: (0, 0)),
            pl.BlockSpec((ROWB, O), lambda i: (i, 0)),
            pl.BlockSpec((1, O), lambda i: (0, 0)),
            pl.BlockSpec((1, 1), lambda i: (0, 0)),
        ],
        out_specs=pl.BlockSpec((1, 1), lambda i: (0, 0)),
        out_shape=jax.ShapeDtypeStruct((1, 1), jnp.float32),
        scratch_shapes=[pltpu.VMEM((8, O), jnp.float32)],
    )(hh, stats, bn_g, bn_b, mixWT, mixb, hprev, conv_w, conv_b)


def _prep(p, F, Fp, O, Fin_store):
    Mw, Mb, Uw, Ub = p["M_w"], p["M_b"], p["U_w"], p["U_b"]

    def z(a, r, c):
        return jnp.pad(a, ((0, r - a.shape[0]), (0, c - a.shape[1])))

    return dict(
        W1=z(Mw[:, :F].T, Fin_store, Fp),
        W2=z(Mw[:, F:2 * F].T, Fin_store, Fp),
        b1=jnp.pad(Mb, (0, Fp - F)).reshape(1, Fp),
        m0=jnp.pad(Mw[:, 2 * F], (0, Fp - F)),
        m1=jnp.pad(Mw[:, 2 * F + 1], (0, Fp - F)),
        Uh=z(Uw[:, :F].T, Fin_store, O),
        Umean=z(Uw[:, F:2 * F].T, Fp, O),
        Umx=z(Uw[:, 2 * F:3 * F].T, Fp, O),
        Us=z(Uw[:, 3 * F:4 * F].T, Fp, O),
        Ustd=z(Uw[:, 4 * F:5 * F].T, Fp, O),
        Ub=Ub.reshape(1, O),
        bn_g=p["bn_g"].reshape(1, O), bn_b=p["bn_b"].reshape(1, O),
        mixWT=p["mix_w"].T, mixb=p["mix_b"].reshape(1, O))


def _layer(h, pp, Fp, O, srcp, dlocp, e0p, e1p, bstart, bcount, deg2d,
           residual, ns_s=None, out_width=None, final=None):
    A, Bm = _tc_pre(h, pp["W1"], pp["b1"], pp["W2"], Fp)
    SUMf, SQf, MXf, CNT = _sc_agg(Fp, A, srcp, dlocp, e0p, e1p,
                                  bstart, bcount, pp["m0"], pp["m1"])
    SUM = SUMf.reshape(NPAD, Fp)
    SQ = SQf.reshape(NPAD, Fp)
    MX = MXf.reshape(NPAD, Fp)
    if deg2d is None:
        deg2d = CNT.astype(jnp.float32).reshape(NPAD, 1)
    hh, stats = _tc_post1(h, Bm, SUM, SQ, MX, deg2d,
                          pp["Uh"], pp["Umean"], pp["Umx"], pp["Us"],
                          pp["Ustd"], pp["Ub"], O)
    if final is not None:
        conv_w, conv_b = final
        out = _tc_post2_final(hh, stats, h, pp["bn_g"], pp["bn_b"],
                              pp["mixWT"], pp["mixb"], conv_w, conv_b)
        return out, deg2d
    hn = _tc_post2(hh, stats, h if residual else None, pp["bn_g"], pp["bn_b"],
                   pp["mixWT"], pp["mixb"], O, residual,
                   out_width=out_width, ns_s=ns_s)
    return hn, deg2d


def kernel(node_attr, edge_attr, edge_index, params):
    src = edge_index[0]
    dst = edge_index[1]
    e0 = edge_attr[:, 0]
    e1 = edge_attr[:, 1]
    ns = node_attr[:, :24]
    ns_s = node_attr[:, 24:25]

    hist = _sc_hist(dst).reshape(NT, 256)
    cur0, bstart8, bcount8 = _tc_offsets(hist)
    bstart = bstart8[0]
    bcount = bcount8[0]
    srcp, dlocp, e0p, e1p = _sc_scatter(dst, src, e0, e1, cur0.reshape(-1))

    P = params
    pp1 = _prep(P["p1"], 24, 32, 64, 24)
    pp2 = _prep(P["p2"], 64, 64, 64, 64)
    pp3 = _prep(P["p3"], 64, 64, 8, 64)
    pp4 = _prep(P["p01"], 12, 16, 64, 16)
    pp5 = _prep(P["p02"], 64, 64, 64, 64)

    common = (srcp, dlocp, e0p, e1p, bstart, bcount)
    h, deg2d = _layer(ns, pp1, 32, 64, *common, None, False)
    h, _ = _layer(h, pp2, 64, 64, *common, deg2d, True)
    h, _ = _layer(h, pp3, 64, 8, *common, deg2d, False,
                  ns_s=ns_s, out_width=16)
    h, _ = _layer(h, pp4, 16, 64, *common, deg2d, False)
    out, _ = _layer(h, pp5, 64, 64, *common, deg2d, True,
                    final=(P["conv_w"], P["conv_b"].reshape(1, 1)))
    return out.reshape(1, 1, 1)


# speedup vs baseline: 4.4188x; 2.9280x over previous
import functools

import jax
import jax.numpy as jnp
from jax import lax
from jax.experimental import pallas as pl
from jax.experimental.pallas import tpu as pltpu
from jax.experimental.pallas import tpu_sc as plsc

N_NODES = 50000
N_EDGES = 800000
NT = 32
NPB = 256
NB = (N_NODES + NPB - 1) // NPB
NPAD = NB * NPB
EPT = N_EDGES // NT
E_ALLOC = N_EDGES + NT * NB * 16 + 1024
DUMP = E_ALLOC - 1
ROWB = 1000
NROWB = N_NODES // ROWB

_SC_PARAMS = pltpu.CompilerParams(
    needs_layout_passes=False, use_tc_tiling_on_sc=False)


def _mesh():
    return plsc.VectorSubcoreMesh(core_axis_name="c", subcore_axis_name="s")


def _iota16():
    return lax.broadcasted_iota(jnp.int32, (16,), 0)


def _full16(v):
    return jnp.full((16,), v, jnp.int32)


def _lane(vec, i):
    return vec.at[_full16(i)].get(mode="promise_in_bounds")


def _extract(vec, lane_idx):
    return jnp.sum(jnp.where(_iota16() == lane_idx, vec, 0))


def _sc_hist(dst):
    @functools.partial(
        pl.kernel,
        out_type=jax.ShapeDtypeStruct((NT * 256,), jnp.int32),
        mesh=_mesh(),
        compiler_params=_SC_PARAMS,
        scratch_types=[
            pltpu.VMEM((EPT + 32,), jnp.int32),
            pltpu.VMEM((256,), jnp.int32),
            pltpu.SMEM((256,), jnp.int32),
        ],
    )
    def k(dst_h, out_h, dbuf, histv, cs):
        IOTA = _iota16()
        wid = lax.axis_index("s") * 2 + lax.axis_index("c")
        pltpu.sync_copy(dst_h.at[pl.ds(pl.multiple_of(wid * EPT, 8), EPT)], dbuf.at[pl.ds(0, EPT)])

        def zero(i, _):
            cs[i] = jnp.int32(0)
            return 0

        lax.fori_loop(0, 256, zero, 0)

        def grp(g, _):
            d16 = plsc.load_gather(dbuf, [_full16(0) + g * 16 + IOTA])
            b16 = lax.shift_right_logical(d16, 8)
            for i in range(16):
                bi = b16[i]
                cs[bi] = cs[bi] + 1
            return 0

        lax.fori_loop(0, EPT // 16, grp, 0)
        nt_ = EPT % 16
        if nt_:
            d16 = plsc.load_gather(dbuf, [_full16((EPT // 16) * 16) + IOTA])
            b16 = lax.shift_right_logical(d16, 8)
            for i in range(nt_):
                bi = b16[i]
                cs[bi] = cs[bi] + 1

        def wb(g, _):
            v = jnp.zeros((16,), jnp.int32)
            for i in range(16):
                v = jnp.where(IOTA == i, cs[g * 16 + i], v)
            plsc.store_scatter(histv, [_full16(0) + g * 16 + IOTA], v)
            return 0

        lax.fori_loop(0, 16, wb, 0)
        pltpu.sync_copy(histv, out_h.at[pl.ds(pl.multiple_of(wid * 256, 8), 256)])

    return k(dst)


def _sc_offsets(hist):
    @functools.partial(
        pl.kernel,
        out_type=(
            jax.ShapeDtypeStruct((NT * 256,), jnp.int32),
            jax.ShapeDtypeStruct((256,), jnp.int32),
            jax.ShapeDtypeStruct((256,), jnp.int32),
        ),
        mesh=_mesh(),
        compiler_params=_SC_PARAMS,
        scratch_types=[
            pltpu.VMEM((NT * 256,), jnp.int32),
            pltpu.VMEM((NT * 256,), jnp.int32),
            pltpu.VMEM((256,), jnp.int32),
            pltpu.VMEM((256,), jnp.int32),
        ],
    )
    def k(hist_h, cur_h, bs_h, bc_h, hbuf, cbuf, s1v, bbv):
        IOTA = _iota16()
        wid = lax.axis_index("s") * 2 + lax.axis_index("c")

        @pl.when(wid == 0)
        def _():
            pltpu.sync_copy(hist_h, hbuf)

            def colgrp(bg, _):
                col = _full16(0) + bg * 16 + IOTA
                acc = jnp.zeros((16,), jnp.int32)
                for t in range(NT):
                    plsc.store_scatter(cbuf, [col + t * 256], acc)
                    v = plsc.load_gather(hbuf, [col + t * 256])
                    hp = lax.bitwise_and(v + 15, jnp.int32(-16))
                    acc = acc + hp
                plsc.store_scatter(s1v, [col], acc)
                return 0

            lax.fori_loop(0, 16, colgrp, 0)

            def scang(g, carry):
                col = _full16(0) + g * 16 + IOTA
                v = plsc.load_gather(s1v, [col])
                c = plsc.cumsum(v)
                plsc.store_scatter(bbv, [col], c - v + carry)
                return carry + c[15]

            lax.fori_loop(0, 16, scang, jnp.int32(0))

            def addbb(bg, _):
                col = _full16(0) + bg * 16 + IOTA
                bbx = plsc.load_gather(bbv, [col])
                for t in range(NT):
                    v = plsc.load_gather(cbuf, [col + t * 256])
                    plsc.store_scatter(cbuf, [col + t * 256], v + bbx)
                return 0

            lax.fori_loop(0, 16, addbb, 0)

            pltpu.sync_copy(cbuf, cur_h)
            pltpu.sync_copy(bbv, bs_h)
            pltpu.sync_copy(s1v, bc_h)

    return k(hist)


EC = 3200
NFULL = EPT // EC
ETAIL = EPT - NFULL * EC
TG = ETAIL // 16
TREM = ETAIL - TG * 16


def _sc_scatter(dst, src, e0, e1, cur0):
    @functools.partial(
        pl.kernel,
        out_type=jax.ShapeDtypeStruct((E_ALLOC, 16), jnp.int32),
        mesh=_mesh(),
        compiler_params=_SC_PARAMS,
        scratch_types=[
            pltpu.VMEM((EC,), jnp.int32),
            pltpu.VMEM((EC,), jnp.int32),
            pltpu.VMEM((EC,), jnp.float32),
            pltpu.VMEM((EC,), jnp.float32),
            pltpu.VMEM((25, 128, 16), jnp.int32),
            pltpu.VMEM((25, 128), jnp.int32),
            pltpu.VMEM((256,), jnp.int32),
            pltpu.VMEM((32, 128), jnp.int32),
            pltpu.VMEM((128, 16), jnp.int32),
            pltpu.SMEM((256,), jnp.int32),
            pltpu.SemaphoreType.DMA,
        ],
    )
    def k(dst_h, src_h, e0_h, e1_h, cur0_h, rec_h,
          dstg, srcg, e0g, e1g, recg, posb, curv, padpos, padv, cs, sem):
        IOTA = _iota16()
        wid = lax.axis_index("s") * 2 + lax.axis_index("c")
        base_e = wid * EPT
        pltpu.sync_copy(cur0_h.at[pl.ds(pl.multiple_of(wid * 256, 8), 256)],
                        curv)

        def stage_cur(g, _):
            v = plsc.load_gather(curv, [_full16(0) + g * 16 + IOTA])
            for i in range(16):
                cs[g * 16 + i] = v[i]
            return 0

        lax.fori_loop(0, 16, stage_cur, 0)

        def do_group(gbase, nlanes):
            rows16 = _full16(0) + gbase + IOTA
            d16 = plsc.load_gather(dstg, [rows16])
            s16 = plsc.load_gather(srcg, [rows16])
            ev0 = plsc.bitcast(plsc.load_gather(e0g, [rows16]), jnp.int32)
            ev1 = plsc.bitcast(plsc.load_gather(e1g, [rows16]), jnp.int32)
            b16 = lax.shift_right_logical(d16, 8)
            dl16 = lax.bitwise_and(d16, jnp.int32(255))
            prow0 = lax.div(gbase, jnp.int32(128))
            lrow = lax.rem(gbase, jnp.int32(128)) + IOTA
            plsc.store_scatter(recg, [_full16(0) + prow0, lrow, _full16(0)], s16)
            plsc.store_scatter(recg, [_full16(0) + prow0, lrow, _full16(1)], dl16)
            plsc.store_scatter(recg, [_full16(0) + prow0, lrow, _full16(2)], ev0)
            plsc.store_scatter(recg, [_full16(0) + prow0, lrow, _full16(3)], ev1)
            posv = _full16(DUMP)
            valid = _iota16() < nlanes
            pvs = [cs[b16[i]] for i in range(nlanes)]
            for i in range(nlanes):
                bi_s = _lane(b16, i)
                eq = (b16 == bi_s) & valid
                before = eq & (IOTA < i)
                after = eq & (IOTA > i)
                dup = plsc.all_reduce_population_count(before)[0]
                nafter = plsc.all_reduce_population_count(after)[0]
                pos_i = pvs[i] + dup
                posv = jnp.where(IOTA == i, pos_i, posv)

                @pl.when(nafter == 0)
                def _():
                    cs[b16[i]] = pos_i + 1

            prow = lax.div(gbase, jnp.int32(128))
            pcol = lax.rem(gbase, jnp.int32(128))
            plsc.store_scatter(posb, [_full16(0) + prow, pcol + IOTA], posv)

        def flush():
            cps = []
            for r in range(25):
                cps.append(pltpu.async_copy(recg.at[r],
                                            rec_h.at[posb.at[r]], sem))
                if len(cps) == 4:
                    for cp in cps:
                        cp.wait()
                    cps = []
            for cp in cps:
                cp.wait()

        def chunk(cidx, _):
            off = pl.multiple_of(base_e + cidx * EC, 8)
            pltpu.sync_copy(dst_h.at[pl.ds(off, EC)], dstg)
            pltpu.sync_copy(src_h.at[pl.ds(off, EC)], srcg)
            pltpu.sync_copy(e0_h.at[pl.ds(off, EC)], e0g)
            pltpu.sync_copy(e1_h.at[pl.ds(off, EC)], e1g)

            def grp(g, _):
                do_group(g * 16, 16)
                return 0

            lax.fori_loop(0, EC // 16, grp, 0)
            flush()
            return 0

        lax.fori_loop(0, NFULL, chunk, 0)

        off = pl.multiple_of(base_e + NFULL * EC, 8)
        pltpu.sync_copy(dst_h.at[pl.ds(off, ETAIL)], dstg.at[pl.ds(0, ETAIL)])
        pltpu.sync_copy(src_h.at[pl.ds(off, ETAIL)], srcg.at[pl.ds(0, ETAIL)])
        pltpu.sync_copy(e0_h.at[pl.ds(off, ETAIL)], e0g.at[pl.ds(0, ETAIL)])
        pltpu.sync_copy(e1_h.at[pl.ds(off, ETAIL)], e1g.at[pl.ds(0, ETAIL)])
        dv = _full16(DUMP)

        def pfill(r, _):
            for c in range(8):
                plsc.store_scatter(posb, [_full16(0) + r, c * 16 + IOTA], dv)
            return 0

        lax.fori_loop(0, 25, pfill, 0)

        def grp_t(g, _):
            do_group(g * 16, 16)
            return 0

        lax.fori_loop(0, TG, grp_t, 0)
        if TREM:
            do_group(TG * 16, TREM)
        flush()

        zi = jnp.zeros((16,), jnp.int32)
        for jb in range(8):
            rows16 = _full16(jb * 16) + IOTA
            plsc.store_scatter(padv, [rows16, _full16(0)], zi)
            plsc.store_scatter(padv, [rows16, _full16(1)], _full16(256))
            plsc.store_scatter(padv, [rows16, _full16(2)], zi)
            plsc.store_scatter(padv, [rows16, _full16(3)], zi)

        def padb(b, _):
            p0 = cs[b]
            npad = lax.bitwise_and(-p0, jnp.int32(15))
            pv = jnp.where(IOTA < npad, p0 + IOTA, _full16(DUMP))
            prow = lax.shift_right_logical(b, 3)
            pcol = lax.rem(b, jnp.int32(8)) * 16
            plsc.store_scatter(padpos, [_full16(0) + prow, pcol + IOTA], pv)
            return 0

        lax.fori_loop(0, 256, padb, 0)
        for r in range(32):
            pltpu.async_copy(padv, rec_h.at[padpos.at[r]], sem).wait()

    return k(dst, src, e0, e1, cur0)


def _sc_agg(Fp, A, rec, bstart, bcount, m0, m1):
    NCH = Fp // 16

    @functools.partial(
        pl.kernel,
        out_type=(
            jax.ShapeDtypeStruct((NPAD * Fp,), jnp.float32),
            jax.ShapeDtypeStruct((NPAD * Fp,), jnp.float32),
            jax.ShapeDtypeStruct((NPAD * Fp,), jnp.float32),
            jax.ShapeDtypeStruct((NPAD,), jnp.int32),
        ),
        mesh=_mesh(),
        compiler_params=_SC_PARAMS,
        scratch_types=[
            pltpu.VMEM((257 * Fp,), jnp.float32),
            pltpu.VMEM((257 * Fp,), jnp.float32),
            pltpu.VMEM((257 * Fp,), jnp.float32),
            pltpu.VMEM((256,), jnp.int32),
            pltpu.VMEM((128, 16), jnp.int32),
            pltpu.VMEM((128,), jnp.int32),
            pltpu.VMEM((128, Fp), jnp.float32),
            pltpu.VMEM((256,), jnp.int32),
            pltpu.VMEM((256,), jnp.int32),
            pltpu.VMEM((Fp,), jnp.float32),
            pltpu.VMEM((Fp,), jnp.float32),
            pltpu.SMEM((272,), jnp.int32),
            pltpu.SemaphoreType.DMA,
        ],
    )
    def k(A_h, rec_h, bs_h, bc_h, m0_h, m1_h,
          sum_h, sq_h, mx_h, cnt_h,
          accs, accq, accm, cntv, recbuf, idxbuf, rows,
          bsv, bcv, m0v, m1v, cnts, sem):
        IOTA = _iota16()
        wid = lax.axis_index("s") * 2 + lax.axis_index("c")
        pltpu.sync_copy(bs_h, bsv)
        pltpu.sync_copy(bc_h, bcv)
        pltpu.sync_copy(m0_h, m0v)
        pltpu.sync_copy(m1_h, m1v)
        m0c = [plsc.load_gather(m0v, [IOTA + c * 16]) for c in range(NCH)]
        m1c = [plsc.load_gather(m1v, [IOTA + c * 16]) for c in range(NCH)]

        def bucket(kk, _):
            b = wid + kk * 32

            @pl.when(b < NB)
            def _():
                zf = jnp.zeros((16,), jnp.float32)
                ninf = jnp.full((16,), -3.0e38, jnp.float32)

                def zrow(r, _):
                    rb = _full16(0) + r * Fp
                    for c in range(NCH):
                        fi = rb + (IOTA + c * 16)
                        plsc.store_scatter(accs, [fi], zf)
                        plsc.store_scatter(accq, [fi], zf)
                        plsc.store_scatter(accm, [fi], ninf)
                    return 0

                lax.fori_loop(0, 257, zrow, 0)

                def czero(i, _):
                    cnts[i] = jnp.int32(0)
                    return 0

                lax.fori_loop(0, 272, czero, 0)

                cbase = lax.shift_left(lax.shift_right_logical(b, 4), 4)
                lane = lax.bitwise_and(b, jnp.int32(15))
                start = _extract(
                    plsc.load_gather(bsv, [_full16(0) + cbase + IOTA]), lane)
                cnt = _extract(
                    plsc.load_gather(bcv, [_full16(0) + cbase + IOTA]), lane)
                ngroups = lax.div(cnt + 127, jnp.int32(128))

                def group(g, _):
                    off = pl.multiple_of(start + g * 128, 8)
                    pltpu.sync_copy(rec_h.at[pl.ds(off, 128)], recbuf)
                    vg = cnt - g * 128
                    for jb in range(8):
                        j16 = _full16(jb * 16) + IOTA
                        sv = plsc.load_gather(recbuf, [j16, _full16(0)])
                        sv = jnp.where(j16 < vg, sv, 0)
                        plsc.store_scatter(idxbuf, [j16], sv)
                    pltpu.async_copy(A_h.at[idxbuf], rows, sem).wait()
                    nsub = jnp.minimum(jnp.int32(8),
                                       lax.div(vg, jnp.int32(16)))

                    def sub(s, _):
                        sb = s * 16
                        r16 = _full16(0) + sb + IOTA
                        dl16 = plsc.load_gather(recbuf, [r16, _full16(1)])
                        e016 = plsc.bitcast(
                            plsc.load_gather(recbuf, [r16, _full16(2)]),
                            jnp.float32)
                        e116 = plsc.bitcast(
                            plsc.load_gather(recbuf, [r16, _full16(3)]),
                            jnp.float32)
                        for i in range(16):
                            di = dl16[i]
                            cnts[di] = cnts[di] + 1
                            dli = _lane(dl16, i)
                            e0i = e016.at[_full16(i)].get(
                                mode="promise_in_bounds")
                            e1i = e116.at[_full16(i)].get(
                                mode="promise_in_bounds")
                            rb = dli * Fp
                            for c in range(NCH):
                                row = plsc.load_gather(
                                    rows,
                                    [_full16(0) + sb + i, IOTA + c * 16])
                                p = row + e0i * m0c[c] + e1i * m1c[c]
                                fi = rb + (IOTA + c * 16)
                                plsc.addupdate_scatter(accs, [fi], p)
                                plsc.addupdate_scatter(accq, [fi], p * p)
                                cm = plsc.load_gather(accm, [fi])
                                plsc.store_scatter(accm, [fi],
                                                   jnp.maximum(cm, p))
                        return 0

                    lax.fori_loop(0, nsub, sub, 0)
                    return 0

                lax.fori_loop(0, ngroups, group, 0)

                pltpu.sync_copy(accs.at[pl.ds(0, 256 * Fp)],
                                sum_h.at[pl.ds(pl.multiple_of(b * (256 * Fp), 8), 256 * Fp)])
                pltpu.sync_copy(accq.at[pl.ds(0, 256 * Fp)],
                                sq_h.at[pl.ds(pl.multiple_of(b * (256 * Fp), 8), 256 * Fp)])
                pltpu.sync_copy(accm.at[pl.ds(0, 256 * Fp)],
                                mx_h.at[pl.ds(pl.multiple_of(b * (256 * Fp), 8), 256 * Fp)])

                def cwb(g, _):
                    v = jnp.zeros((16,), jnp.int32)
                    for i in range(16):
                        v = jnp.where(IOTA == i, cnts[g * 16 + i], v)
                    plsc.store_scatter(cntv, [_full16(0) + g * 16 + IOTA], v)
                    return 0

                lax.fori_loop(0, 16, cwb, 0)
                pltpu.sync_copy(cntv,
                                cnt_h.at[pl.ds(pl.multiple_of(b * 256, 8), 256)])

            return 0

        lax.fori_loop(0, 7, bucket, 0)

    return k(A, rec, bstart, bcount, m0, m1)


def _dot(a, b):
    return jax.lax.dot_general(a, b, (((1,), (0,)), ((), ())),
                               preferred_element_type=jnp.float32)


def _tc_pre(h, W1, b1, W2, Fp):
    Fin = h.shape[1]

    def body(h_ref, w1_ref, b1_ref, w2_ref, a_ref, b_ref):
        hb = h_ref[...]
        a_ref[...] = _dot(hb, w1_ref[...]) + b1_ref[...]
        b_ref[...] = _dot(hb, w2_ref[...])

    return pl.pallas_call(
        body,
        grid=(NROWB,),
        in_specs=[
            pl.BlockSpec((ROWB, Fin), lambda i: (i, 0)),
            pl.BlockSpec((Fin, Fp), lambda i: (0, 0)),
            pl.BlockSpec((1, Fp), lambda i: (0, 0)),
            pl.BlockSpec((Fin, Fp), lambda i: (0, 0)),
        ],
        out_specs=(
            pl.BlockSpec((ROWB, Fp), lambda i: (i, 0)),
            pl.BlockSpec((ROWB, Fp), lambda i: (i, 0)),
        ),
        out_shape=(
            jax.ShapeDtypeStruct((N_NODES, Fp), jnp.float32),
            jax.ShapeDtypeStruct((N_NODES, Fp), jnp.float32),
        ),
    )(h, W1, b1, W2)


def _tc_post1(h, Bm, SUM, SQ, MX, deg2d, Uw_h, Uw_mean, Uw_mx, Uw_s, Uw_std,
              Ub, O):
    Fin = h.shape[1]
    Fp = Bm.shape[1]

    def body(h_ref, b_ref, sum_ref, sq_ref, mx_ref, deg_ref,
             uh_ref, um_ref, ux_ref, us_ref, ud_ref, ub_ref,
             hh_ref, st_ref, sacc):
        i = pl.program_id(0)

        @pl.when(i == 0)
        def _():
            sacc[...] = jnp.zeros((8, O), jnp.float32)

        hb = h_ref[...]
        B = b_ref[...]
        SA = sum_ref[...]
        SQb = sq_ref[...]
        MXb = mx_ref[...]
        deg = deg_ref[...]
        degc = jnp.maximum(deg, 1.0)
        s = SA + deg * B
        s2 = SQb + 2.0 * B * SA + deg * B * B
        mx = jnp.where(deg > 0, B + MXb, 0.0)
        mean = s / degc
        var = jnp.maximum(s2 / degc - mean * mean, 0.0)
        std = jnp.sqrt(var + 1e-30)
        hh = (_dot(hb, uh_ref[...]) + _dot(mean, um_ref[...])
              + _dot(mx, ux_ref[...]) + _dot(s, us_ref[...])
              + _dot(std, ud_ref[...]) + ub_ref[...])
        hh = hh * (1.0 / N_NODES)
        hh_ref[...] = hh
        sacc[0:1, :] = sacc[0:1, :] + jnp.sum(hh, axis=0, keepdims=True)
        sacc[1:2, :] = sacc[1:2, :] + jnp.sum(hh * hh, axis=0, keepdims=True)

        @pl.when(i == NROWB - 1)
        def _():
            st_ref[...] = sacc[...]

    return pl.pallas_call(
        body,
        grid=(NROWB,),
        in_specs=[
            pl.BlockSpec((ROWB, Fin), lambda i: (i, 0)),
            pl.BlockSpec((ROWB, Fp), lambda i: (i, 0)),
            pl.BlockSpec((ROWB, Fp), lambda i: (i, 0)),
            pl.BlockSpec((ROWB, Fp), lambda i: (i, 0)),
            pl.BlockSpec((ROWB, Fp), lambda i: (i, 0)),
            pl.BlockSpec((ROWB, 1), lambda i: (i, 0)),
            pl.BlockSpec((Fin, O), lambda i: (0, 0)),
            pl.BlockSpec((Fp, O), lambda i: (0, 0)),
            pl.BlockSpec((Fp, O), lambda i: (0, 0)),
            pl.BlockSpec((Fp, O), lambda i: (0, 0)),
            pl.BlockSpec((Fp, O), lambda i: (0, 0)),
            pl.BlockSpec((1, O), lambda i: (0, 0)),
        ],
        out_specs=(
            pl.BlockSpec((ROWB, O), lambda i: (i, 0)),
            pl.BlockSpec((8, O), lambda i: (0, 0)),
        ),
        out_shape=(
            jax.ShapeDtypeStruct((N_NODES, O), jnp.float32),
            jax.ShapeDtypeStruct((8, O), jnp.float32),
        ),
        scratch_shapes=[pltpu.VMEM((8, O), jnp.float32)],
    )(h, Bm, SUM, SQ, MX, deg2d, Uw_h, Uw_mean, Uw_mx, Uw_s, Uw_std, Ub)


def _tc_post2(hh, stats, hprev, bn_g, bn_b, mixWT, mixb, O, residual,
              out_width=None, ns_s=None):
    ow = out_width or O
    in_specs = [
        pl.BlockSpec((ROWB, O), lambda i: (i, 0)),
        pl.BlockSpec((8, O), lambda i: (0, 0)),
        pl.BlockSpec((1, O), lambda i: (0, 0)),
        pl.BlockSpec((1, O), lambda i: (0, 0)),
        pl.BlockSpec((O, O), lambda i: (0, 0)),
        pl.BlockSpec((1, O), lambda i: (0, 0)),
    ]
    args = [hh, stats, bn_g, bn_b, mixWT, mixb]
    if residual:
        in_specs.append(pl.BlockSpec((ROWB, O), lambda i: (i, 0)))
        args.append(hprev)
    if ns_s is not None:
        in_specs.append(pl.BlockSpec((ROWB, 1), lambda i: (i, 0)))
        args.append(ns_s)

    def body(*refs):
        hh_ref, st_ref, g_ref, bb_ref, mw_ref, mb_ref = refs[:6]
        rest = list(refs[6:-1])
        o_ref = refs[-1]
        hb = hh_ref[...]
        st = st_ref[...]
        m = st[0:1, :] * (1.0 / N_NODES)
        msq = st[1:2, :] * (1.0 / N_NODES)
        inv = jax.lax.rsqrt(msq - m * m + 1e-5)
        xn = (hb - m) * inv * g_ref[...] + bb_ref[...]
        out = _dot(xn, mw_ref[...]) + mb_ref[...]
        out = jnp.where(out >= 0, out, 0.01 * out)
        if residual:
            out = out + rest.pop(0)[...]
        out = jnp.maximum(out, 0.0)
        if ns_s is not None:
            nss = rest.pop(0)[...]
            out = jnp.concatenate(
                [out, jnp.broadcast_to(nss, (ROWB, 4)),
                 jnp.zeros((ROWB, ow - O - 4), jnp.float32)], axis=1)
        o_ref[...] = out

    return pl.pallas_call(
        body,
        grid=(NROWB,),
        in_specs=in_specs,
        out_specs=pl.BlockSpec((ROWB, ow), lambda i: (i, 0)),
        out_shape=jax.ShapeDtypeStruct((N_NODES, ow), jnp.float32),
    )(*args)


def _tc_post2_final(hh, stats, hprev, bn_g, bn_b, mixWT, mixb, conv_w, conv_b):
    O = 64

    def body(hh_ref, st_ref, g_ref, bb_ref, mw_ref, mb_ref, hp_ref,
             cw_ref, cb_ref, o_ref, gm):
        i = pl.program_id(0)

        @pl.when(i == 0)
        def _():
            gm[...] = jnp.full((8, O), -3.0e38, jnp.float32)

        hb = hh_ref[...]
        st = st_ref[...]
        m = st[0:1, :] * (1.0 / N_NODES)
        msq = st[1:2, :] * (1.0 / N_NODES)
        inv = jax.lax.rsqrt(msq - m * m + 1e-5)
        xn = (hb - m) * inv * g_ref[...] + bb_ref[...]
        out = _dot(xn, mw_ref[...]) + mb_ref[...]
        out = jnp.where(out >= 0, out, 0.01 * out)
        out = jnp.maximum(out + hp_ref[...], 0.0)
        bm = jnp.max(out, axis=0, keepdims=True)
        gm[0:1, :] = jnp.maximum(gm[0:1, :], bm)

        @pl.when(i == NROWB - 1)
        def _():
            val = jnp.sum(gm[0:1, :] * cw_ref[...]) + jnp.sum(cb_ref[...])
            o_ref[...] = jnp.broadcast_to(jnp.maximum(val, 0.0), (1, 1))

    return pl.pallas_call(
        body,
        grid=(NROWB,),
        in_specs=[
            pl.BlockSpec((ROWB, O), lambda i: (i, 0)),
            pl.BlockSpec((8, O), lambda i: (0, 0)),
            pl.BlockSpec((1, O), lambda i: (0, 0)),
            pl.BlockSpec((1, O), lambda i: (0, 0)),
            pl.BlockSpec((O, O), lambda i: (0, 0)),
            pl.BlockSpec((1, O), lambda i: (0, 0)),
            pl.BlockSpec((ROWB, O), lambda i: (i, 0)),
            pl.BlockSpec((1, O), lambda i: (0, 0)),
            pl.BlockSpec((1, 1), lambda i: (0, 0)),
        ],
        out_specs=pl.BlockSpec((1, 1), lambda i: (0, 0)),
        out_shape=jax.ShapeDtypeStruct((1, 1), jnp.float32),
        scratch_shapes=[pltpu.VMEM((8, O), jnp.float32)],
    )(hh, stats, bn_g, bn_b, mixWT, mixb, hprev, conv_w, conv_b)


def _prep(p, F, Fp, O, Fin_store):
    Mw, Mb, Uw, Ub = p["M_w"], p["M_b"], p["U_w"], p["U_b"]

    def z(a, r, c):
        return jnp.pad(a, ((0, r - a.shape[0]), (0, c - a.shape[1])))

    return dict(
        W1=z(Mw[:, :F].T, Fin_store, Fp),
        W2=z(Mw[:, F:2 * F].T, Fin_store, Fp),
        b1=jnp.pad(Mb, (0, Fp - F)).reshape(1, Fp),
        m0=jnp.pad(Mw[:, 2 * F], (0, Fp - F)),
        m1=jnp.pad(Mw[:, 2 * F + 1], (0, Fp - F)),
        Uh=z(Uw[:, :F].T, Fin_store, O),
        Umean=z(Uw[:, F:2 * F].T, Fp, O),
        Umx=z(Uw[:, 2 * F:3 * F].T, Fp, O),
        Us=z(Uw[:, 3 * F:4 * F].T, Fp, O),
        Ustd=z(Uw[:, 4 * F:5 * F].T, Fp, O),
        Ub=Ub.reshape(1, O),
        bn_g=p["bn_g"].reshape(1, O), bn_b=p["bn_b"].reshape(1, O),
        mixWT=p["mix_w"].T, mixb=p["mix_b"].reshape(1, O))


def _layer(h, pp, Fp, O, rec, bstart, bcount, deg2d,
           residual, ns_s=None, out_width=None, final=None):
    A, Bm = _tc_pre(h, pp["W1"], pp["b1"], pp["W2"], Fp)
    SUMf, SQf, MXf, CNT = _sc_agg(Fp, A, rec,
                                  bstart, bcount, pp["m0"], pp["m1"])
    SUM = SUMf.reshape(NPAD, Fp)
    SQ = SQf.reshape(NPAD, Fp)
    MX = MXf.reshape(NPAD, Fp)
    if deg2d is None:
        deg2d = CNT.astype(jnp.float32).reshape(NPAD, 1)
    hh, stats = _tc_post1(h, Bm, SUM, SQ, MX, deg2d,
                          pp["Uh"], pp["Umean"], pp["Umx"], pp["Us"],
                          pp["Ustd"], pp["Ub"], O)
    if final is not None:
        conv_w, conv_b = final
        out = _tc_post2_final(hh, stats, h, pp["bn_g"], pp["bn_b"],
                              pp["mixWT"], pp["mixb"], conv_w, conv_b)
        return out, deg2d
    hn = _tc_post2(hh, stats, h if residual else None, pp["bn_g"], pp["bn_b"],
                   pp["mixWT"], pp["mixb"], O, residual,
                   out_width=out_width, ns_s=ns_s)
    return hn, deg2d


def kernel(node_attr, edge_attr, edge_index, params):
    src = edge_index[0]
    dst = edge_index[1]
    e0 = edge_attr[:, 0]
    e1 = edge_attr[:, 1]
    ns = node_attr[:, :24]
    ns_s = node_attr[:, 24:25]

    hist = _sc_hist(dst)
    cur0, bstart, bcount = _sc_offsets(hist)
    rec = _sc_scatter(dst, src, e0, e1, cur0)

    P = params
    pp1 = _prep(P["p1"], 24, 32, 64, 24)
    pp2 = _prep(P["p2"], 64, 64, 64, 64)
    pp3 = _prep(P["p3"], 64, 64, 8, 64)
    pp4 = _prep(P["p01"], 12, 16, 64, 16)
    pp5 = _prep(P["p02"], 64, 64, 64, 64)

    common = (rec, bstart, bcount)
    h, deg2d = _layer(ns, pp1, 32, 64, *common, None, False)
    h, _ = _layer(h, pp2, 64, 64, *common, deg2d, True)
    h, _ = _layer(h, pp3, 64, 8, *common, deg2d, False,
                  ns_s=ns_s, out_width=16)
    h, _ = _layer(h, pp4, 16, 64, *common, deg2d, False)
    out, _ = _layer(h, pp5, 64, 64, *common, deg2d, True,
                    final=(P["conv_w"], P["conv_b"].reshape(1, 1)))
    return out.reshape(1, 1, 1)


# speedup vs baseline: 4.4709x; 1.0118x over previous
import functools

import jax
import jax.numpy as jnp
from jax import lax
from jax.experimental import pallas as pl
from jax.experimental.pallas import tpu as pltpu
from jax.experimental.pallas import tpu_sc as plsc

N_NODES = 50000
N_EDGES = 800000
NT = 32
NPB = 256
NB = (N_NODES + NPB - 1) // NPB
NPAD = NB * NPB
EPT = N_EDGES // NT
E_ALLOC = N_EDGES + NT * NB * 16 + 1024
DUMP = E_ALLOC - 1
ROWB = 1000
NROWB = N_NODES // ROWB

_SC_PARAMS = pltpu.CompilerParams(
    needs_layout_passes=False, use_tc_tiling_on_sc=False)


def _mesh():
    return plsc.VectorSubcoreMesh(core_axis_name="c", subcore_axis_name="s")


def _iota16():
    return lax.broadcasted_iota(jnp.int32, (16,), 0)


def _full16(v):
    return jnp.full((16,), v, jnp.int32)


def _lane(vec, i):
    return vec.at[_full16(i)].get(mode="promise_in_bounds")


def _extract(vec, lane_idx):
    return jnp.sum(jnp.where(_iota16() == lane_idx, vec, 0))


def _sc_hist(dst):
    @functools.partial(
        pl.kernel,
        out_type=jax.ShapeDtypeStruct((NT * 256,), jnp.int32),
        mesh=_mesh(),
        compiler_params=_SC_PARAMS,
        scratch_types=[
            pltpu.VMEM((EPT + 32,), jnp.int32),
            pltpu.VMEM((256,), jnp.int32),
            pltpu.SMEM((256,), jnp.int32),
        ],
    )
    def k(dst_h, out_h, dbuf, histv, cs):
        IOTA = _iota16()
        wid = lax.axis_index("s") * 2 + lax.axis_index("c")
        pltpu.sync_copy(dst_h.at[pl.ds(pl.multiple_of(wid * EPT, 8), EPT)], dbuf.at[pl.ds(0, EPT)])

        def zero(i, _):
            cs[i] = jnp.int32(0)
            return 0

        lax.fori_loop(0, 256, zero, 0)

        def grp(g, _):
            d16 = plsc.load_gather(dbuf, [_full16(0) + g * 16 + IOTA])
            b16 = lax.shift_right_logical(d16, 8)
            for i in range(16):
                bi = b16[i]
                cs[bi] = cs[bi] + 1
            return 0

        lax.fori_loop(0, EPT // 16, grp, 0)
        nt_ = EPT % 16
        if nt_:
            d16 = plsc.load_gather(dbuf, [_full16((EPT // 16) * 16) + IOTA])
            b16 = lax.shift_right_logical(d16, 8)
            for i in range(nt_):
                bi = b16[i]
                cs[bi] = cs[bi] + 1

        def wb(g, _):
            v = jnp.zeros((16,), jnp.int32)
            for i in range(16):
                v = jnp.where(IOTA == i, cs[g * 16 + i], v)
            plsc.store_scatter(histv, [_full16(0) + g * 16 + IOTA], v)
            return 0

        lax.fori_loop(0, 16, wb, 0)
        pltpu.sync_copy(histv, out_h.at[pl.ds(pl.multiple_of(wid * 256, 8), 256)])

    return k(dst)


def _sc_offsets(hist):
    @functools.partial(
        pl.kernel,
        out_type=(
            jax.ShapeDtypeStruct((NT * 256,), jnp.int32),
            jax.ShapeDtypeStruct((256,), jnp.int32),
            jax.ShapeDtypeStruct((256,), jnp.int32),
        ),
        mesh=_mesh(),
        compiler_params=_SC_PARAMS,
        scratch_types=[
            pltpu.VMEM((NT * 256,), jnp.int32),
            pltpu.VMEM((NT * 256,), jnp.int32),
            pltpu.VMEM((256,), jnp.int32),
            pltpu.VMEM((256,), jnp.int32),
        ],
    )
    def k(hist_h, cur_h, bs_h, bc_h, hbuf, cbuf, s1v, bbv):
        IOTA = _iota16()
        wid = lax.axis_index("s") * 2 + lax.axis_index("c")

        @pl.when(wid == 0)
        def _():
            pltpu.sync_copy(hist_h, hbuf)

            def colgrp(bg, _):
                col = _full16(0) + bg * 16 + IOTA
                acc = jnp.zeros((16,), jnp.int32)
                for t in range(NT):
                    plsc.store_scatter(cbuf, [col + t * 256], acc)
                    v = plsc.load_gather(hbuf, [col + t * 256])
                    hp = lax.bitwise_and(v + 15, jnp.int32(-16))
                    acc = acc + hp
                plsc.store_scatter(s1v, [col], acc)
                return 0

            lax.fori_loop(0, 16, colgrp, 0)

            def scang(g, carry):
                col = _full16(0) + g * 16 + IOTA
                v = plsc.load_gather(s1v, [col])
                c = plsc.cumsum(v)
                plsc.store_scatter(bbv, [col], c - v + carry)
                return carry + c[15]

            lax.fori_loop(0, 16, scang, jnp.int32(0))

            def addbb(bg, _):
                col = _full16(0) + bg * 16 + IOTA
                bbx = plsc.load_gather(bbv, [col])
                for t in range(NT):
                    v = plsc.load_gather(cbuf, [col + t * 256])
                    plsc.store_scatter(cbuf, [col + t * 256], v + bbx)
                return 0

            lax.fori_loop(0, 16, addbb, 0)

            pltpu.sync_copy(cbuf, cur_h)
            pltpu.sync_copy(bbv, bs_h)
            pltpu.sync_copy(s1v, bc_h)

    return k(hist)


EC = 3200
NFULL = EPT // EC
ETAIL = EPT - NFULL * EC
TG = ETAIL // 16
TREM = ETAIL - TG * 16


def _sc_scatter(dst, src, e0, e1, cur0):
    @functools.partial(
        pl.kernel,
        out_type=jax.ShapeDtypeStruct((E_ALLOC, 16), jnp.int32),
        mesh=_mesh(),
        compiler_params=_SC_PARAMS,
        scratch_types=[
            pltpu.VMEM((EC,), jnp.int32),
            pltpu.VMEM((EC,), jnp.int32),
            pltpu.VMEM((EC,), jnp.float32),
            pltpu.VMEM((EC,), jnp.float32),
            pltpu.VMEM((25, 128, 16), jnp.int32),
            pltpu.VMEM((25, 128), jnp.int32),
            pltpu.VMEM((256,), jnp.int32),
            pltpu.VMEM((32, 128), jnp.int32),
            pltpu.VMEM((128, 16), jnp.int32),
            pltpu.SMEM((256,), jnp.int32),
            pltpu.SemaphoreType.DMA,
        ],
    )
    def k(dst_h, src_h, e0_h, e1_h, cur0_h, rec_h,
          dstg, srcg, e0g, e1g, recg, posb, curv, padpos, padv, cs, sem):
        IOTA = _iota16()
        wid = lax.axis_index("s") * 2 + lax.axis_index("c")
        base_e = wid * EPT
        pltpu.sync_copy(cur0_h.at[pl.ds(pl.multiple_of(wid * 256, 8), 256)],
                        curv)

        def stage_cur(g, _):
            v = plsc.load_gather(curv, [_full16(0) + g * 16 + IOTA])
            for i in range(16):
                cs[g * 16 + i] = v[i]
            return 0

        lax.fori_loop(0, 16, stage_cur, 0)

        def do_group(gbase, nlanes):
            rows16 = _full16(0) + gbase + IOTA
            d16 = plsc.load_gather(dstg, [rows16])
            s16 = plsc.load_gather(srcg, [rows16])
            ev0 = plsc.bitcast(plsc.load_gather(e0g, [rows16]), jnp.int32)
            ev1 = plsc.bitcast(plsc.load_gather(e1g, [rows16]), jnp.int32)
            b16 = lax.shift_right_logical(d16, 8)
            dl16 = lax.bitwise_and(d16, jnp.int32(255))
            prow0 = lax.div(gbase, jnp.int32(128))
            lrow = lax.rem(gbase, jnp.int32(128)) + IOTA
            plsc.store_scatter(recg, [_full16(0) + prow0, lrow, _full16(0)], s16)
            plsc.store_scatter(recg, [_full16(0) + prow0, lrow, _full16(1)], dl16)
            plsc.store_scatter(recg, [_full16(0) + prow0, lrow, _full16(2)], ev0)
            plsc.store_scatter(recg, [_full16(0) + prow0, lrow, _full16(3)], ev1)
            posv = _full16(DUMP)
            valid = _iota16() < nlanes
            pvs = [cs[b16[i]] for i in range(nlanes)]
            for i in range(nlanes):
                bi_s = _lane(b16, i)
                eq = (b16 == bi_s) & valid
                before = eq & (IOTA < i)
                after = eq & (IOTA > i)
                dup = plsc.all_reduce_population_count(before)[0]
                nafter = plsc.all_reduce_population_count(after)[0]
                pos_i = pvs[i] + dup
                posv = jnp.where(IOTA == i, pos_i, posv)

                @pl.when(nafter == 0)
                def _():
                    cs[b16[i]] = pos_i + 1

            prow = lax.div(gbase, jnp.int32(128))
            pcol = lax.rem(gbase, jnp.int32(128))
            plsc.store_scatter(posb, [_full16(0) + prow, pcol + IOTA], posv)

        def flush():
            cps = []
            for r in range(25):
                cps.append(pltpu.async_copy(recg.at[r],
                                            rec_h.at[posb.at[r]], sem))
                if len(cps) == 4:
                    for cp in cps:
                        cp.wait()
                    cps = []
            for cp in cps:
                cp.wait()

        def chunk(cidx, _):
            off = pl.multiple_of(base_e + cidx * EC, 8)
            pltpu.sync_copy(dst_h.at[pl.ds(off, EC)], dstg)
            pltpu.sync_copy(src_h.at[pl.ds(off, EC)], srcg)
            pltpu.sync_copy(e0_h.at[pl.ds(off, EC)], e0g)
            pltpu.sync_copy(e1_h.at[pl.ds(off, EC)], e1g)

            def grp(g, _):
                do_group(g * 16, 16)
                return 0

            lax.fori_loop(0, EC // 16, grp, 0)
            flush()
            return 0

        lax.fori_loop(0, NFULL, chunk, 0)

        off = pl.multiple_of(base_e + NFULL * EC, 8)
        pltpu.sync_copy(dst_h.at[pl.ds(off, ETAIL)], dstg.at[pl.ds(0, ETAIL)])
        pltpu.sync_copy(src_h.at[pl.ds(off, ETAIL)], srcg.at[pl.ds(0, ETAIL)])
        pltpu.sync_copy(e0_h.at[pl.ds(off, ETAIL)], e0g.at[pl.ds(0, ETAIL)])
        pltpu.sync_copy(e1_h.at[pl.ds(off, ETAIL)], e1g.at[pl.ds(0, ETAIL)])
        dv = _full16(DUMP)

        def pfill(r, _):
            for c in range(8):
                plsc.store_scatter(posb, [_full16(0) + r, c * 16 + IOTA], dv)
            return 0

        lax.fori_loop(0, 25, pfill, 0)

        def grp_t(g, _):
            do_group(g * 16, 16)
            return 0

        lax.fori_loop(0, TG, grp_t, 0)
        if TREM:
            do_group(TG * 16, TREM)
        flush()

        zi = jnp.zeros((16,), jnp.int32)
        for jb in range(8):
            rows16 = _full16(jb * 16) + IOTA
            plsc.store_scatter(padv, [rows16, _full16(0)], zi)
            plsc.store_scatter(padv, [rows16, _full16(1)], _full16(256))
            plsc.store_scatter(padv, [rows16, _full16(2)], zi)
            plsc.store_scatter(padv, [rows16, _full16(3)], zi)

        def padb(b, _):
            p0 = cs[b]
            npad = lax.bitwise_and(-p0, jnp.int32(15))
            pv = jnp.where(IOTA < npad, p0 + IOTA, _full16(DUMP))
            prow = lax.shift_right_logical(b, 3)
            pcol = lax.rem(b, jnp.int32(8)) * 16
            plsc.store_scatter(padpos, [_full16(0) + prow, pcol + IOTA], pv)
            return 0

        lax.fori_loop(0, 256, padb, 0)
        for r in range(32):
            pltpu.async_copy(padv, rec_h.at[padpos.at[r]], sem).wait()

    return k(dst, src, e0, e1, cur0)


def _sc_agg(Fp, A, rec, bstart, bcount, m0, m1):
    NCH = Fp // 16

    @functools.partial(
        pl.kernel,
        out_type=(
            jax.ShapeDtypeStruct((NPAD * Fp,), jnp.float32),
            jax.ShapeDtypeStruct((NPAD * Fp,), jnp.float32),
            jax.ShapeDtypeStruct((NPAD * Fp,), jnp.float32),
            jax.ShapeDtypeStruct((NPAD,), jnp.int32),
        ),
        mesh=_mesh(),
        compiler_params=_SC_PARAMS,
        scratch_types=[
            pltpu.VMEM((257 * Fp,), jnp.float32),
            pltpu.VMEM((257 * Fp,), jnp.float32),
            pltpu.VMEM((257 * Fp,), jnp.float32),
            pltpu.VMEM((256,), jnp.int32),
            pltpu.VMEM((256, 16), jnp.int32),
            pltpu.VMEM((256,), jnp.int32),
            pltpu.VMEM((256, Fp), jnp.float32),
            pltpu.VMEM((256,), jnp.int32),
            pltpu.VMEM((256,), jnp.int32),
            pltpu.VMEM((Fp,), jnp.float32),
            pltpu.VMEM((Fp,), jnp.float32),
            pltpu.SMEM((272,), jnp.int32),
            pltpu.SemaphoreType.DMA,
        ],
    )
    def k(A_h, rec_h, bs_h, bc_h, m0_h, m1_h,
          sum_h, sq_h, mx_h, cnt_h,
          accs, accq, accm, cntv, recbuf, idxbuf, rows,
          bsv, bcv, m0v, m1v, cnts, sem):
        IOTA = _iota16()
        wid = lax.axis_index("s") * 2 + lax.axis_index("c")
        pltpu.sync_copy(bs_h, bsv)
        pltpu.sync_copy(bc_h, bcv)
        pltpu.sync_copy(m0_h, m0v)
        pltpu.sync_copy(m1_h, m1v)
        m0c = [plsc.load_gather(m0v, [IOTA + c * 16]) for c in range(NCH)]
        m1c = [plsc.load_gather(m1v, [IOTA + c * 16]) for c in range(NCH)]

        def bucket(kk, _):
            b = wid + kk * 32

            @pl.when(b < NB)
            def _():
                zf = jnp.zeros((16,), jnp.float32)
                ninf = jnp.full((16,), -3.0e38, jnp.float32)

                def zrow(r, _):
                    rb = _full16(0) + r * Fp
                    for c in range(NCH):
                        fi = rb + (IOTA + c * 16)
                        plsc.store_scatter(accs, [fi], zf)
                        plsc.store_scatter(accq, [fi], zf)
                        plsc.store_scatter(accm, [fi], ninf)
                    return 0

                lax.fori_loop(0, 257, zrow, 0)

                def czero(i, _):
                    cnts[i] = jnp.int32(0)
                    return 0

                lax.fori_loop(0, 272, czero, 0)

                cbase = lax.shift_left(lax.shift_right_logical(b, 4), 4)
                lane = lax.bitwise_and(b, jnp.int32(15))
                start = _extract(
                    plsc.load_gather(bsv, [_full16(0) + cbase + IOTA]), lane)
                cnt = _extract(
                    plsc.load_gather(bcv, [_full16(0) + cbase + IOTA]), lane)
                ngroups = lax.div(cnt + 255, jnp.int32(256))

                def group(g, _):
                    off = pl.multiple_of(start + g * 256, 8)
                    pltpu.sync_copy(rec_h.at[pl.ds(off, 256)], recbuf)
                    vg = cnt - g * 256
                    for jb in range(16):
                        j16 = _full16(jb * 16) + IOTA
                        sv = plsc.load_gather(recbuf, [j16, _full16(0)])
                        sv = jnp.where(j16 < vg, sv, 0)
                        plsc.store_scatter(idxbuf, [j16], sv)
                    cp0 = pltpu.async_copy(A_h.at[idxbuf.at[pl.ds(0, 128)]],
                                           rows.at[pl.ds(0, 128)], sem)
                    cp1 = pltpu.async_copy(A_h.at[idxbuf.at[pl.ds(128, 128)]],
                                           rows.at[pl.ds(128, 128)], sem)
                    cp0.wait()
                    cp1.wait()
                    nsub = jnp.minimum(jnp.int32(16),
                                       lax.div(vg, jnp.int32(16)))

                    def sub(s, _):
                        sb = s * 16
                        r16 = _full16(0) + sb + IOTA
                        dl16 = plsc.load_gather(recbuf, [r16, _full16(1)])
                        e016 = plsc.bitcast(
                            plsc.load_gather(recbuf, [r16, _full16(2)]),
                            jnp.float32)
                        e116 = plsc.bitcast(
                            plsc.load_gather(recbuf, [r16, _full16(3)]),
                            jnp.float32)
                        for i in range(16):
                            di = dl16[i]
                            cnts[di] = cnts[di] + 1
                            dli = _lane(dl16, i)
                            e0i = e016.at[_full16(i)].get(
                                mode="promise_in_bounds")
                            e1i = e116.at[_full16(i)].get(
                                mode="promise_in_bounds")
                            rb = dli * Fp
                            for c in range(NCH):
                                row = plsc.load_gather(
                                    rows,
                                    [_full16(0) + sb + i, IOTA + c * 16])
                                p = row + e0i * m0c[c] + e1i * m1c[c]
                                fi = rb + (IOTA + c * 16)
                                plsc.addupdate_scatter(accs, [fi], p)
                                plsc.addupdate_scatter(accq, [fi], p * p)
                                cm = plsc.load_gather(accm, [fi])
                                plsc.store_scatter(accm, [fi],
                                                   jnp.maximum(cm, p))
                        return 0

                    lax.fori_loop(0, nsub, sub, 0)
                    return 0

                lax.fori_loop(0, ngroups, group, 0)

                pltpu.sync_copy(accs.at[pl.ds(0, 256 * Fp)],
                                sum_h.at[pl.ds(pl.multiple_of(b * (256 * Fp), 8), 256 * Fp)])
                pltpu.sync_copy(accq.at[pl.ds(0, 256 * Fp)],
                                sq_h.at[pl.ds(pl.multiple_of(b * (256 * Fp), 8), 256 * Fp)])
                pltpu.sync_copy(accm.at[pl.ds(0, 256 * Fp)],
                                mx_h.at[pl.ds(pl.multiple_of(b * (256 * Fp), 8), 256 * Fp)])

                def cwb(g, _):
                    v = jnp.zeros((16,), jnp.int32)
                    for i in range(16):
                        v = jnp.where(IOTA == i, cnts[g * 16 + i], v)
                    plsc.store_scatter(cntv, [_full16(0) + g * 16 + IOTA], v)
                    return 0

                lax.fori_loop(0, 16, cwb, 0)
                pltpu.sync_copy(cntv,
                                cnt_h.at[pl.ds(pl.multiple_of(b * 256, 8), 256)])

            return 0

        lax.fori_loop(0, 7, bucket, 0)

    return k(A, rec, bstart, bcount, m0, m1)


def _dot(a, b):
    return jax.lax.dot_general(a, b, (((1,), (0,)), ((), ())),
                               preferred_element_type=jnp.float32)


def _tc_pre(h, W1, b1, W2, Fp):
    Fin = h.shape[1]

    def body(h_ref, w1_ref, b1_ref, w2_ref, a_ref, b_ref):
        hb = h_ref[...]
        a_ref[...] = _dot(hb, w1_ref[...]) + b1_ref[...]
        b_ref[...] = _dot(hb, w2_ref[...])

    return pl.pallas_call(
        body,
        grid=(NROWB,),
        in_specs=[
            pl.BlockSpec((ROWB, Fin), lambda i: (i, 0)),
            pl.BlockSpec((Fin, Fp), lambda i: (0, 0)),
            pl.BlockSpec((1, Fp), lambda i: (0, 0)),
            pl.BlockSpec((Fin, Fp), lambda i: (0, 0)),
        ],
        out_specs=(
            pl.BlockSpec((ROWB, Fp), lambda i: (i, 0)),
            pl.BlockSpec((ROWB, Fp), lambda i: (i, 0)),
        ),
        out_shape=(
            jax.ShapeDtypeStruct((N_NODES, Fp), jnp.float32),
            jax.ShapeDtypeStruct((N_NODES, Fp), jnp.float32),
        ),
    )(h, W1, b1, W2)


def _tc_post1(h, Bm, SUM, SQ, MX, deg2d, Uw_h, Uw_mean, Uw_mx, Uw_s, Uw_std,
              Ub, O):
    Fin = h.shape[1]
    Fp = Bm.shape[1]

    def body(h_ref, b_ref, sum_ref, sq_ref, mx_ref, deg_ref,
             uh_ref, um_ref, ux_ref, us_ref, ud_ref, ub_ref,
             hh_ref, st_ref, sacc):
        i = pl.program_id(0)

        @pl.when(i == 0)
        def _():
            sacc[...] = jnp.zeros((8, O), jnp.float32)

        hb = h_ref[...]
        B = b_ref[...]
        SA = sum_ref[...]
        SQb = sq_ref[...]
        MXb = mx_ref[...]
        deg = deg_ref[...]
        degc = jnp.maximum(deg, 1.0)
        s = SA + deg * B
        s2 = SQb + 2.0 * B * SA + deg * B * B
        mx = jnp.where(deg > 0, B + MXb, 0.0)
        mean = s / degc
        var = jnp.maximum(s2 / degc - mean * mean, 0.0)
        std = jnp.sqrt(var + 1e-30)
        hh = (_dot(hb, uh_ref[...]) + _dot(mean, um_ref[...])
              + _dot(mx, ux_ref[...]) + _dot(s, us_ref[...])
              + _dot(std, ud_ref[...]) + ub_ref[...])
        hh = hh * (1.0 / N_NODES)
        hh_ref[...] = hh
        sacc[0:1, :] = sacc[0:1, :] + jnp.sum(hh, axis=0, keepdims=True)
        sacc[1:2, :] = sacc[1:2, :] + jnp.sum(hh * hh, axis=0, keepdims=True)

        @pl.when(i == NROWB - 1)
        def _():
            st_ref[...] = sacc[...]

    return pl.pallas_call(
        body,
        grid=(NROWB,),
        in_specs=[
            pl.BlockSpec((ROWB, Fin), lambda i: (i, 0)),
            pl.BlockSpec((ROWB, Fp), lambda i: (i, 0)),
            pl.BlockSpec((ROWB, Fp), lambda i: (i, 0)),
            pl.BlockSpec((ROWB, Fp), lambda i: (i, 0)),
            pl.BlockSpec((ROWB, Fp), lambda i: (i, 0)),
            pl.BlockSpec((ROWB, 1), lambda i: (i, 0)),
            pl.BlockSpec((Fin, O), lambda i: (0, 0)),
            pl.BlockSpec((Fp, O), lambda i: (0, 0)),
            pl.BlockSpec((Fp, O), lambda i: (0, 0)),
            pl.BlockSpec((Fp, O), lambda i: (0, 0)),
            pl.BlockSpec((Fp, O), lambda i: (0, 0)),
            pl.BlockSpec((1, O), lambda i: (0, 0)),
        ],
        out_specs=(
            pl.BlockSpec((ROWB, O), lambda i: (i, 0)),
            pl.BlockSpec((8, O), lambda i: (0, 0)),
        ),
        out_shape=(
            jax.ShapeDtypeStruct((N_NODES, O), jnp.float32),
            jax.ShapeDtypeStruct((8, O), jnp.float32),
        ),
        scratch_shapes=[pltpu.VMEM((8, O), jnp.float32)],
    )(h, Bm, SUM, SQ, MX, deg2d, Uw_h, Uw_mean, Uw_mx, Uw_s, Uw_std, Ub)


def _tc_post2(hh, stats, hprev, bn_g, bn_b, mixWT, mixb, O, residual,
              out_width=None, ns_s=None):
    ow = out_width or O
    in_specs = [
        pl.BlockSpec((ROWB, O), lambda i: (i, 0)),
        pl.BlockSpec((8, O), lambda i: (0, 0)),
        pl.BlockSpec((1, O), lambda i: (0, 0)),
        pl.BlockSpec((1, O), lambda i: (0, 0)),
        pl.BlockSpec((O, O), lambda i: (0, 0)),
        pl.BlockSpec((1, O), lambda i: (0, 0)),
    ]
    args = [hh, stats, bn_g, bn_b, mixWT, mixb]
    if residual:
        in_specs.append(pl.BlockSpec((ROWB, O), lambda i: (i, 0)))
        args.append(hprev)
    if ns_s is not None:
        in_specs.append(pl.BlockSpec((ROWB, 1), lambda i: (i, 0)))
        args.append(ns_s)

    def body(*refs):
        hh_ref, st_ref, g_ref, bb_ref, mw_ref, mb_ref = refs[:6]
        rest = list(refs[6:-1])
        o_ref = refs[-1]
        hb = hh_ref[...]
        st = st_ref[...]
        m = st[0:1, :] * (1.0 / N_NODES)
        msq = st[1:2, :] * (1.0 / N_NODES)
        inv = jax.lax.rsqrt(msq - m * m + 1e-5)
        xn = (hb - m) * inv * g_ref[...] + bb_ref[...]
        out = _dot(xn, mw_ref[...]) + mb_ref[...]
        out = jnp.where(out >= 0, out, 0.01 * out)
        if residual:
            out = out + rest.pop(0)[...]
        out = jnp.maximum(out, 0.0)
        if ns_s is not None:
            nss = rest.pop(0)[...]
            out = jnp.concatenate(
                [out, jnp.broadcast_to(nss, (ROWB, 4)),
                 jnp.zeros((ROWB, ow - O - 4), jnp.float32)], axis=1)
        o_ref[...] = out

    return pl.pallas_call(
        body,
        grid=(NROWB,),
        in_specs=in_specs,
        out_specs=pl.BlockSpec((ROWB, ow), lambda i: (i, 0)),
        out_shape=jax.ShapeDtypeStruct((N_NODES, ow), jnp.float32),
    )(*args)


def _tc_post2_final(hh, stats, hprev, bn_g, bn_b, mixWT, mixb, conv_w, conv_b):
    O = 64

    def body(hh_ref, st_ref, g_ref, bb_ref, mw_ref, mb_ref, hp_ref,
             cw_ref, cb_ref, o_ref, gm):
        i = pl.program_id(0)

        @pl.when(i == 0)
        def _():
            gm[...] = jnp.full((8, O), -3.0e38, jnp.float32)

        hb = hh_ref[...]
        st = st_ref[...]
        m = st[0:1, :] * (1.0 / N_NODES)
        msq = st[1:2, :] * (1.0 / N_NODES)
        inv = jax.lax.rsqrt(msq - m * m + 1e-5)
        xn = (hb - m) * inv * g_ref[...] + bb_ref[...]
        out = _dot(xn, mw_ref[...]) + mb_ref[...]
        out = jnp.where(out >= 0, out, 0.01 * out)
        out = jnp.maximum(out + hp_ref[...], 0.0)
        bm = jnp.max(out, axis=0, keepdims=True)
        gm[0:1, :] = jnp.maximum(gm[0:1, :], bm)

        @pl.when(i == NROWB - 1)
        def _():
            val = jnp.sum(gm[0:1, :] * cw_ref[...]) + jnp.sum(cb_ref[...])
            o_ref[...] = jnp.broadcast_to(jnp.maximum(val, 0.0), (1, 1))

    return pl.pallas_call(
        body,
        grid=(NROWB,),
        in_specs=[
            pl.BlockSpec((ROWB, O), lambda i: (i, 0)),
            pl.BlockSpec((8, O), lambda i: (0, 0)),
            pl.BlockSpec((1, O), lambda i: (0, 0)),
            pl.BlockSpec((1, O), lambda i: (0, 0)),
            pl.BlockSpec((O, O), lambda i: (0, 0)),
            pl.BlockSpec((1, O), lambda i: (0, 0)),
            pl.BlockSpec((ROWB, O), lambda i: (i, 0)),
            pl.BlockSpec((1, O), lambda i: (0, 0)),
            pl.BlockSpec((1, 1), lambda i: (0, 0)),
        ],
        out_specs=pl.BlockSpec((1, 1), lambda i: (0, 0)),
        out_shape=jax.ShapeDtypeStruct((1, 1), jnp.float32),
        scratch_shapes=[pltpu.VMEM((8, O), jnp.float32)],
    )(hh, stats, bn_g, bn_b, mixWT, mixb, hprev, conv_w, conv_b)


def _prep(p, F, Fp, O, Fin_store):
    Mw, Mb, Uw, Ub = p["M_w"], p["M_b"], p["U_w"], p["U_b"]

    def z(a, r, c):
        return jnp.pad(a, ((0, r - a.shape[0]), (0, c - a.shape[1])))

    return dict(
        W1=z(Mw[:, :F].T, Fin_store, Fp),
        W2=z(Mw[:, F:2 * F].T, Fin_store, Fp),
        b1=jnp.pad(Mb, (0, Fp - F)).reshape(1, Fp),
        m0=jnp.pad(Mw[:, 2 * F], (0, Fp - F)),
        m1=jnp.pad(Mw[:, 2 * F + 1], (0, Fp - F)),
        Uh=z(Uw[:, :F].T, Fin_store, O),
        Umean=z(Uw[:, F:2 * F].T, Fp, O),
        Umx=z(Uw[:, 2 * F:3 * F].T, Fp, O),
        Us=z(Uw[:, 3 * F:4 * F].T, Fp, O),
        Ustd=z(Uw[:, 4 * F:5 * F].T, Fp, O),
        Ub=Ub.reshape(1, O),
        bn_g=p["bn_g"].reshape(1, O), bn_b=p["bn_b"].reshape(1, O),
        mixWT=p["mix_w"].T, mixb=p["mix_b"].reshape(1, O))


def _layer(h, pp, Fp, O, rec, bstart, bcount, deg2d,
           residual, ns_s=None, out_width=None, final=None):
    A, Bm = _tc_pre(h, pp["W1"], pp["b1"], pp["W2"], Fp)
    SUMf, SQf, MXf, CNT = _sc_agg(Fp, A, rec,
                                  bstart, bcount, pp["m0"], pp["m1"])
    SUM = SUMf.reshape(NPAD, Fp)
    SQ = SQf.reshape(NPAD, Fp)
    MX = MXf.reshape(NPAD, Fp)
    if deg2d is None:
        deg2d = CNT.astype(jnp.float32).reshape(NPAD, 1)
    hh, stats = _tc_post1(h, Bm, SUM, SQ, MX, deg2d,
                          pp["Uh"], pp["Umean"], pp["Umx"], pp["Us"],
                          pp["Ustd"], pp["Ub"], O)
    if final is not None:
        conv_w, conv_b = final
        out = _tc_post2_final(hh, stats, h, pp["bn_g"], pp["bn_b"],
                              pp["mixWT"], pp["mixb"], conv_w, conv_b)
        return out, deg2d
    hn = _tc_post2(hh, stats, h if residual else None, pp["bn_g"], pp["bn_b"],
                   pp["mixWT"], pp["mixb"], O, residual,
                   out_width=out_width, ns_s=ns_s)
    return hn, deg2d


def kernel(node_attr, edge_attr, edge_index, params):
    src = edge_index[0]
    dst = edge_index[1]
    e0 = edge_attr[:, 0]
    e1 = edge_attr[:, 1]
    ns = node_attr[:, :24]
    ns_s = node_attr[:, 24:25]

    hist = _sc_hist(dst)
    cur0, bstart, bcount = _sc_offsets(hist)
    rec = _sc_scatter(dst, src, e0, e1, cur0)

    P = params
    pp1 = _prep(P["p1"], 24, 32, 64, 24)
    pp2 = _prep(P["p2"], 64, 64, 64, 64)
    pp3 = _prep(P["p3"], 64, 64, 8, 64)
    pp4 = _prep(P["p01"], 12, 16, 64, 16)
    pp5 = _prep(P["p02"], 64, 64, 64, 64)

    common = (rec, bstart, bcount)
    h, deg2d = _layer(ns, pp1, 32, 64, *common, None, False)
    h, _ = _layer(h, pp2, 64, 64, *common, deg2d, True)
    h, _ = _layer(h, pp3, 64, 8, *common, deg2d, False,
                  ns_s=ns_s, out_width=16)
    h, _ = _layer(h, pp4, 16, 64, *common, deg2d, False)
    out, _ = _layer(h, pp5, 64, 64, *common, deg2d, True,
                    final=(P["conv_w"], P["conv_b"].reshape(1, 1)))
    return out.reshape(1, 1, 1)
